# Initial kernel scaffold; baseline (speedup 1.0000x reference)
import jax
import jax.numpy as jnp
from jax.experimental import pallas as pl


def kernel(x, position_enc):
    raise NotImplementedError("write your pallas kernel here")


# speedup vs baseline: 2.8570x; 2.8570x over previous
import jax
import jax.numpy as jnp
from jax.experimental import pallas as pl


_BS = 512


def _body(x_ref, pe_ref, o_ref):
    o_ref[...] = jnp.where(x_ref[...] == 0.0, 0.0, pe_ref[...][None, :, :])


def kernel(x, position_enc):
    B, S, D = x.shape
    pe = position_enc[:S]
    grid = (S // _BS, B)
    return pl.pallas_call(
        _body,
        grid=grid,
        in_specs=[
            pl.BlockSpec((1, _BS, D), lambda s, b: (b, s, 0)),
            pl.BlockSpec((_BS, D), lambda s, b: (s, 0)),
        ],
        out_specs=pl.BlockSpec((1, _BS, D), lambda s, b: (b, s, 0)),
        out_shape=jax.ShapeDtypeStruct((B, S, D), jnp.float32),
    )(x, pe)
